# baseline (speedup 1.0000x reference)
import functools

import jax
import jax.numpy as jnp
from jax import lax
from jax.experimental import pallas as pl
from jax.experimental.pallas import tpu as pltpu
from jax.experimental.pallas import tpu_sc as plsc

N = 10000
D = 128
R = 16
NBASES = 8
E = 320000

NC = 2
NS = 16
NW = NC * NS
CH = 128
CPW = 80
EPW = CH * CPW
E_PAD = NW * EPW
N_PAD = 10240
RPS = N_PAD // NS
DW = 16


def _table_body(coeff_ref, weight_ref, x_ref, out_ref):
    c = coeff_ref[0, 0, :]
    w = weight_ref[...].reshape(NBASES, D * D)
    wr = jnp.dot(c[None, :], w, preferred_element_type=jnp.float32)
    wr = wr.reshape(D, D)
    out_ref[...] = jnp.dot(
        x_ref[...], wr, preferred_element_type=jnp.float32
    ).astype(jnp.bfloat16)


def _compute_table(x, weight, coeff3):
    nb = 5
    blk = N // nb
    return pl.pallas_call(
        _table_body,
        grid=(R, nb),
        in_specs=[
            pl.BlockSpec((1, 1, NBASES), lambda r, b: (r, 0, 0)),
            pl.BlockSpec((NBASES, D, D), lambda r, b: (0, 0, 0)),
            pl.BlockSpec((blk, D), lambda r, b: (b, 0)),
        ],
        out_specs=pl.BlockSpec((blk, D), lambda r, b: (r * nb + b, 0)),
        out_shape=jax.ShapeDtypeStruct((R * N, D), jnp.bfloat16),
    )(coeff3, weight, x)


GG = 8
NG = CPW // GG


def _sc_body(table, gidx2, tgt2, outp, degout,
             idx_v, tgt_v, rows_a, rows_b, zbuf, acc, dacc,
             semg, sem_sa, sem_sb, sem_d):
    c = lax.axis_index("c")
    s = lax.axis_index("s")
    wid = s * NC + c

    zeros32b = jnp.zeros((32,), jnp.bfloat16)

    @pl.loop(0, CH)
    def _zero_rows(r):
        for k in range(D // 32):
            rows_a[r, pl.ds(k * 32, 32)] = zeros32b

    @pl.loop(0, CH)
    def _zero_zbuf(i):
        zbuf[i, :] = jnp.zeros((16,), jnp.float32)

    for i in range(RPS // CH):
        pltpu.sync_copy(rows_a, acc.at[pl.ds(s * RPS + i * CH, CH)])
        pltpu.sync_copy(zbuf, dacc.at[pl.ds(s * RPS + i * CH, CH)])
    plsc.subcore_barrier()

    ones16 = jnp.ones((16,), jnp.float32)

    @pl.loop(0, CH)
    def _ones_zbuf(i):
        zbuf[i, :] = ones16

    @pl.loop(0, NG)
    def _groups(g):
        base = wid * CPW + g * GG
        pltpu.sync_copy(gidx2.at[pl.ds(base, GG)], idx_v)
        pltpu.sync_copy(tgt2.at[pl.ds(base, GG)], tgt_v)
        bufs = (rows_a, rows_b)
        ssems = (sem_sa, sem_sb)
        gather = pltpu.async_copy(table.at[idx_v.at[0]], rows_a, semg)
        pend_scat = [None, None]
        pend_deg = []
        for jj in range(GG):
            p = jj % 2
            gather.wait()
            if jj + 1 < GG:
                q = (jj + 1) % 2
                if pend_scat[q] is not None:
                    pend_scat[q].wait()
                    pend_scat[q] = None
                gather = pltpu.async_copy(
                    table.at[idx_v.at[jj + 1]], bufs[q], semg)
            pend_scat[p] = pltpu.async_copy(
                bufs[p], acc.at[tgt_v.at[jj]], ssems[p], add=True)
            pend_deg.append(pltpu.async_copy(
                zbuf, dacc.at[tgt_v.at[jj]], sem_d, add=True))
        for d in pend_scat:
            if d is not None:
                d.wait()
        for d in pend_deg:
            d.wait()

    plsc.subcore_barrier()

    pltpu.sync_copy(acc.at[pl.ds(s * RPS, RPS)],
                    outp.at[pl.ds(c * N_PAD + s * RPS, RPS)])
    pltpu.sync_copy(dacc.at[pl.ds(s * RPS, RPS)],
                    degout.at[pl.ds(c * N_PAD + s * RPS, RPS)])


_sc_edge_call = pl.kernel(
    _sc_body,
    out_type=[
        jax.ShapeDtypeStruct((NC * N_PAD, D), jnp.bfloat16),
        jax.ShapeDtypeStruct((NC * N_PAD, DW), jnp.float32),
    ],
    mesh=plsc.VectorSubcoreMesh(core_axis_name="c", subcore_axis_name="s"),
    compiler_params=pltpu.CompilerParams(use_tc_tiling_on_sc=False),
    scratch_types=[
        pltpu.VMEM((GG, CH), jnp.int32),
        pltpu.VMEM((GG, CH), jnp.int32),
        pltpu.VMEM((CH, D), jnp.bfloat16),
        pltpu.VMEM((CH, D), jnp.bfloat16),
        pltpu.VMEM((CH, DW), jnp.float32),
        pltpu.VMEM_SHARED((N_PAD, D), jnp.bfloat16),
        pltpu.VMEM_SHARED((N_PAD, DW), jnp.float32),
        pltpu.SemaphoreType.DMA,
        pltpu.SemaphoreType.DMA,
        pltpu.SemaphoreType.DMA,
        pltpu.SemaphoreType.DMA,
    ],
)


BLK = 2048


def _final_body(p_ref, deg_ref, x_ref, sw_ref, b_ref, out_ref):
    psum = (p_ref[0].astype(jnp.float32)
            + p_ref[1].astype(jnp.float32))
    deg = deg_ref[0, :, 0] + deg_ref[1, :, 0]
    deg = jnp.maximum(deg, 1.0)
    self_t = jnp.dot(x_ref[...], sw_ref[...],
                     preferred_element_type=jnp.float32)
    out_ref[...] = psum / deg[:, None] + self_t + b_ref[...]


def _final_call(partial, deg, x, self_weight, bias2):
    return pl.pallas_call(
        _final_body,
        grid=(N_PAD // BLK,),
        in_specs=[
            pl.BlockSpec((NC, BLK, D), lambda b: (0, b, 0)),
            pl.BlockSpec((NC, BLK, DW), lambda b: (0, b, 0)),
            pl.BlockSpec((BLK, D), lambda b: (b, 0)),
            pl.BlockSpec((D, D), lambda b: (0, 0)),
            pl.BlockSpec((1, D), lambda b: (0, 0)),
        ],
        out_specs=pl.BlockSpec((BLK, D), lambda b: (b, 0)),
        out_shape=jax.ShapeDtypeStruct((N, D), jnp.float32),
    )(partial, deg, x, self_weight, bias2)


def kernel(x, edge_index, edge_type, num_nodes, weight, coeff,
           self_weight, bias):
    x = x.astype(jnp.float32)
    src = edge_index[0].astype(jnp.int32)
    tgt = edge_index[1].astype(jnp.int32)
    et = edge_type.astype(jnp.int32)

    pad = E_PAD - E
    ar = jnp.arange(pad, dtype=jnp.int32)
    gidx = et * N + src
    gidx_p = jnp.concatenate([gidx, (ar * 37) % (R * N)])
    tgt_p = jnp.concatenate([tgt, N + ar % (N_PAD - N)])

    table = _compute_table(x, weight.astype(jnp.float32),
                           coeff.astype(jnp.float32).reshape(R, 1, NBASES))
    partial, deg = _sc_edge_call(
        table, gidx_p.reshape(E_PAD // CH, CH), tgt_p.reshape(E_PAD // CH, CH))
    return _final_call(partial.reshape(NC, N_PAD, D),
                       deg.reshape(NC, N_PAD, DW),
                       x, self_weight.astype(jnp.float32),
                       bias.astype(jnp.float32).reshape(1, D))


# speedup vs baseline: 1.3203x; 1.3203x over previous
import functools

import jax
import jax.numpy as jnp
from jax import lax
from jax.experimental import pallas as pl
from jax.experimental.pallas import tpu as pltpu
from jax.experimental.pallas import tpu_sc as plsc

N = 10000
D = 128
R = 16
NBASES = 8
E = 320000

NC = 2
NS = 16
NW = NC * NS
CH = 128
CPW = 80
EPW = CH * CPW
E_PAD = NW * EPW
N_PAD = 10240
RPS = N_PAD // NS
DW = 16


def _table_body(coeff_ref, weight_ref, x_ref, out_ref):
    c = coeff_ref[0, 0, :]
    w = weight_ref[...].reshape(NBASES, D * D)
    wr = jnp.dot(c[None, :], w, preferred_element_type=jnp.float32)
    wr = wr.reshape(D, D).astype(jnp.bfloat16)
    out_ref[...] = jnp.dot(x_ref[...].astype(jnp.bfloat16), wr,
                           preferred_element_type=jnp.float32)


def _compute_table(x, weight, coeff3):
    nb = 5
    blk = N // nb
    return pl.pallas_call(
        _table_body,
        grid=(R, nb),
        in_specs=[
            pl.BlockSpec((1, 1, NBASES), lambda r, b: (r, 0, 0)),
            pl.BlockSpec((NBASES, D, D), lambda r, b: (0, 0, 0)),
            pl.BlockSpec((blk, D), lambda r, b: (b, 0)),
        ],
        out_specs=pl.BlockSpec((blk, D), lambda r, b: (r * nb + b, 0)),
        out_shape=jax.ShapeDtypeStruct((R * N, D), jnp.float32),
    )(coeff3, weight, x)


GG = 8
NG = CPW // GG


def _sc_body(table, gidx2, tgt2, outp, degout,
             idx_v, tgt_v, rows_a, rows_b, zbuf, acc, dacc,
             semg, sem_sa, sem_sb, sem_d):
    c = lax.axis_index("c")
    s = lax.axis_index("s")
    wid = s * NC + c

    zeros16 = jnp.zeros((16,), jnp.float32)

    @pl.loop(0, CH)
    def _zero_rows(r):
        for k in range(D // 16):
            rows_a[r, pl.ds(k * 16, 16)] = zeros16

    @pl.loop(0, CH)
    def _zero_zbuf(i):
        zbuf[i, :] = zeros16

    for i in range(RPS // CH):
        pltpu.sync_copy(rows_a, acc.at[pl.ds(s * RPS + i * CH, CH)])
        pltpu.sync_copy(zbuf, dacc.at[pl.ds(s * RPS + i * CH, CH)])
    plsc.subcore_barrier()

    ones16 = jnp.ones((16,), jnp.float32)

    @pl.loop(0, CH)
    def _ones_zbuf(i):
        zbuf[i, :] = ones16

    @pl.loop(0, NG)
    def _groups(g):
        base = wid * CPW + g * GG
        pltpu.sync_copy(gidx2.at[pl.ds(base, GG)], idx_v)
        pltpu.sync_copy(tgt2.at[pl.ds(base, GG)], tgt_v)
        bufs = (rows_a, rows_b)
        ssems = (sem_sa, sem_sb)
        gather = pltpu.async_copy(table.at[idx_v.at[0]], rows_a, semg)
        pend_scat = [None, None]
        pend_deg = []
        for jj in range(GG):
            p = jj % 2
            gather.wait()
            if jj + 1 < GG:
                q = (jj + 1) % 2
                if pend_scat[q] is not None:
                    pend_scat[q].wait()
                    pend_scat[q] = None
                gather = pltpu.async_copy(
                    table.at[idx_v.at[jj + 1]], bufs[q], semg)
            pend_scat[p] = pltpu.async_copy(
                bufs[p], acc.at[tgt_v.at[jj]], ssems[p], add=True)
            pend_deg.append(pltpu.async_copy(
                zbuf, dacc.at[tgt_v.at[jj]], sem_d, add=True))
        for d in pend_scat:
            if d is not None:
                d.wait()
        for d in pend_deg:
            d.wait()

    plsc.subcore_barrier()

    pltpu.sync_copy(acc.at[pl.ds(s * RPS, RPS)],
                    outp.at[pl.ds(c * N_PAD + s * RPS, RPS)])
    pltpu.sync_copy(dacc.at[pl.ds(s * RPS, RPS)],
                    degout.at[pl.ds(c * N_PAD + s * RPS, RPS)])


_sc_edge_call = pl.kernel(
    _sc_body,
    out_type=[
        jax.ShapeDtypeStruct((NC * N_PAD, D), jnp.float32),
        jax.ShapeDtypeStruct((NC * N_PAD, DW), jnp.float32),
    ],
    mesh=plsc.VectorSubcoreMesh(core_axis_name="c", subcore_axis_name="s"),
    compiler_params=pltpu.CompilerParams(use_tc_tiling_on_sc=False),
    scratch_types=[
        pltpu.VMEM((GG, CH), jnp.int32),
        pltpu.VMEM((GG, CH), jnp.int32),
        pltpu.VMEM((CH, D), jnp.float32),
        pltpu.VMEM((CH, D), jnp.float32),
        pltpu.VMEM((CH, DW), jnp.float32),
        pltpu.VMEM_SHARED((N_PAD, D), jnp.float32),
        pltpu.VMEM_SHARED((N_PAD, DW), jnp.float32),
        pltpu.SemaphoreType.DMA,
        pltpu.SemaphoreType.DMA,
        pltpu.SemaphoreType.DMA,
        pltpu.SemaphoreType.DMA,
    ],
)


BLK = 2048


def _final_body(p_ref, deg_ref, x_ref, sw_ref, b_ref, out_ref):
    psum = p_ref[0] + p_ref[1]
    deg = deg_ref[0, :, 0] + deg_ref[1, :, 0]
    deg = jnp.maximum(deg, 1.0)
    self_t = jnp.dot(x_ref[...], sw_ref[...],
                     preferred_element_type=jnp.float32)
    out_ref[...] = psum / deg[:, None] + self_t + b_ref[...]


def _final_call(partial, deg, x, self_weight, bias2):
    return pl.pallas_call(
        _final_body,
        grid=(N_PAD // BLK,),
        in_specs=[
            pl.BlockSpec((NC, BLK, D), lambda b: (0, b, 0)),
            pl.BlockSpec((NC, BLK, DW), lambda b: (0, b, 0)),
            pl.BlockSpec((BLK, D), lambda b: (b, 0)),
            pl.BlockSpec((D, D), lambda b: (0, 0)),
            pl.BlockSpec((1, D), lambda b: (0, 0)),
        ],
        out_specs=pl.BlockSpec((BLK, D), lambda b: (b, 0)),
        out_shape=jax.ShapeDtypeStruct((N, D), jnp.float32),
    )(partial, deg, x, self_weight, bias2)


def kernel(x, edge_index, edge_type, num_nodes, weight, coeff,
           self_weight, bias):
    x = x.astype(jnp.float32)
    src = edge_index[0].astype(jnp.int32)
    tgt = edge_index[1].astype(jnp.int32)
    et = edge_type.astype(jnp.int32)

    pad = E_PAD - E
    ar = jnp.arange(pad, dtype=jnp.int32)
    gidx = et * N + src
    gidx_p = jnp.concatenate([gidx, (ar * 37) % (R * N)])
    tgt_p = jnp.concatenate([tgt, N + ar % (N_PAD - N)])

    table = _compute_table(x, weight.astype(jnp.float32),
                           coeff.astype(jnp.float32).reshape(R, 1, NBASES))
    partial, deg = _sc_edge_call(
        table, gidx_p.reshape(E_PAD // CH, CH), tgt_p.reshape(E_PAD // CH, CH))
    return _final_call(partial.reshape(NC, N_PAD, D),
                       deg.reshape(NC, N_PAD, DW),
                       x, self_weight.astype(jnp.float32),
                       bias.astype(jnp.float32).reshape(1, D))


# speedup vs baseline: 1.6425x; 1.2440x over previous
import functools

import jax
import jax.numpy as jnp
from jax import lax
from jax.experimental import pallas as pl
from jax.experimental.pallas import tpu as pltpu
from jax.experimental.pallas import tpu_sc as plsc

N = 10000
D = 128
R = 16
NBASES = 8
E = 320000

NC = 2
NS = 16
NW = NC * NS
CH = 128
CPW = 80
EPW = CH * CPW
E_PAD = NW * EPW
N_PAD = 10240
RPS = N_PAD // NS
DW = 16


def _table_body(coeff_ref, weight_ref, x_ref, out_ref):
    w = weight_ref[...].reshape(NBASES, D * D)
    xb = x_ref[...].astype(jnp.bfloat16)
    for r in range(R):
        c = coeff_ref[r, 0, :]
        wr = jnp.dot(c[None, :], w, preferred_element_type=jnp.float32)
        wr = wr.reshape(D, D).astype(jnp.bfloat16)
        out_ref[r] = jnp.dot(xb, wr, preferred_element_type=jnp.float32)


def _compute_table(x, weight, coeff3):
    nb = 5
    blk = N // nb
    return pl.pallas_call(
        _table_body,
        grid=(nb,),
        in_specs=[
            pl.BlockSpec((R, 1, NBASES), lambda b: (0, 0, 0)),
            pl.BlockSpec((NBASES, D, D), lambda b: (0, 0, 0)),
            pl.BlockSpec((blk, D), lambda b: (b, 0)),
        ],
        out_specs=pl.BlockSpec((R, blk, D), lambda b: (0, b, 0)),
        out_shape=jax.ShapeDtypeStruct((R, N, D), jnp.float32),
    )(coeff3, weight, x)


GG = 8
NG = CPW // GG


def _sc_body(table, gidx2, tgt2, outp, degout,
             idx_v, tgt_v, rows_a, rows_b, zbuf, acc, dacc,
             semg, sem_sa, sem_sb, sem_d):
    c = lax.axis_index("c")
    s = lax.axis_index("s")
    wid = s * NC + c

    zeros16 = jnp.zeros((16,), jnp.float32)

    @pl.loop(0, CH)
    def _zero_rows(r):
        for k in range(D // 16):
            rows_a[r, pl.ds(k * 16, 16)] = zeros16

    @pl.loop(0, CH // 2)
    def _zero_zbuf(i):
        zbuf[pl.ds(2 * i, 2), :] = jnp.zeros((2, DW), jnp.int16)

    for i in range(RPS // CH):
        pltpu.sync_copy(rows_a, acc.at[pl.ds(s * RPS + i * CH, CH)])
        pltpu.sync_copy(zbuf, dacc.at[pl.ds(s * RPS + i * CH, CH)])
    plsc.subcore_barrier()

    @pl.loop(0, CH // 2)
    def _ones_zbuf(i):
        zbuf[pl.ds(2 * i, 2), :] = jnp.ones((2, DW), jnp.int16)

    @pl.loop(0, NG)
    def _groups(g):
        base = wid * CPW + g * GG
        pltpu.sync_copy(gidx2.at[pl.ds(base, GG)], idx_v)
        pltpu.sync_copy(tgt2.at[pl.ds(base, GG)], tgt_v)
        bufs = (rows_a, rows_b)
        ssems = (sem_sa, sem_sb)
        gather = pltpu.async_copy(table.at[idx_v.at[0]], rows_a, semg)
        pend_scat = [None, None]
        pend_deg = []
        for jj in range(GG):
            p = jj % 2
            gather.wait()
            if jj + 1 < GG:
                q = (jj + 1) % 2
                if pend_scat[q] is not None:
                    pend_scat[q].wait()
                    pend_scat[q] = None
                gather = pltpu.async_copy(
                    table.at[idx_v.at[jj + 1]], bufs[q], semg)
            pend_scat[p] = pltpu.async_copy(
                bufs[p], acc.at[tgt_v.at[jj]], ssems[p], add=True)
            pend_deg.append(pltpu.async_copy(
                zbuf, dacc.at[tgt_v.at[jj]], sem_d, add=True))
        for d in pend_scat:
            if d is not None:
                d.wait()
        for d in pend_deg:
            d.wait()

    plsc.subcore_barrier()

    pltpu.sync_copy(acc.at[pl.ds(s * RPS, RPS)],
                    outp.at[pl.ds(c * N_PAD + s * RPS, RPS)])
    pltpu.sync_copy(dacc.at[pl.ds(s * RPS, RPS)],
                    degout.at[pl.ds(c * N_PAD + s * RPS, RPS)])


_sc_edge_call = pl.kernel(
    _sc_body,
    out_type=[
        jax.ShapeDtypeStruct((NC * N_PAD, D), jnp.float32),
        jax.ShapeDtypeStruct((NC * N_PAD, DW), jnp.int16),
    ],
    mesh=plsc.VectorSubcoreMesh(core_axis_name="c", subcore_axis_name="s"),
    compiler_params=pltpu.CompilerParams(use_tc_tiling_on_sc=False),
    scratch_types=[
        pltpu.VMEM((GG, CH), jnp.int32),
        pltpu.VMEM((GG, CH), jnp.int32),
        pltpu.VMEM((CH, D), jnp.float32),
        pltpu.VMEM((CH, D), jnp.float32),
        pltpu.VMEM((CH, DW), jnp.int16),
        pltpu.VMEM_SHARED((N_PAD, D), jnp.float32),
        pltpu.VMEM_SHARED((N_PAD, DW), jnp.int16),
        pltpu.SemaphoreType.DMA,
        pltpu.SemaphoreType.DMA,
        pltpu.SemaphoreType.DMA,
        pltpu.SemaphoreType.DMA,
    ],
)


BLK = 2048


def _final_body(p_ref, deg_ref, x_ref, sw_ref, b_ref, out_ref):
    psum = p_ref[0] + p_ref[1]
    deg = (deg_ref[0, :, 0] + deg_ref[1, :, 0]).astype(jnp.float32)
    deg = jnp.maximum(deg, 1.0)
    self_t = jnp.dot(x_ref[...], sw_ref[...],
                     preferred_element_type=jnp.float32)
    out_ref[...] = psum / deg[:, None] + self_t + b_ref[...]


def _final_call(partial, deg, x, self_weight, bias2):
    return pl.pallas_call(
        _final_body,
        grid=(N_PAD // BLK,),
        in_specs=[
            pl.BlockSpec((NC, BLK, D), lambda b: (0, b, 0)),
            pl.BlockSpec((NC, BLK, DW), lambda b: (0, b, 0)),
            pl.BlockSpec((BLK, D), lambda b: (b, 0)),
            pl.BlockSpec((D, D), lambda b: (0, 0)),
            pl.BlockSpec((1, D), lambda b: (0, 0)),
        ],
        out_specs=pl.BlockSpec((BLK, D), lambda b: (b, 0)),
        out_shape=jax.ShapeDtypeStruct((N, D), jnp.float32),
    )(partial, deg, x, self_weight, bias2)


def kernel(x, edge_index, edge_type, num_nodes, weight, coeff,
           self_weight, bias):
    x = x.astype(jnp.float32)
    src = edge_index[0].astype(jnp.int32)
    tgt = edge_index[1].astype(jnp.int32)
    et = edge_type.astype(jnp.int32)

    pad = E_PAD - E
    ar = jnp.arange(pad, dtype=jnp.int32)
    gidx = et * N + src
    gidx_p = jnp.concatenate([gidx, (ar * 37) % (R * N)])
    tgt_p = jnp.concatenate([tgt, N + ar % (N_PAD - N)])

    table = _compute_table(x, weight.astype(jnp.float32),
                           coeff.astype(jnp.float32).reshape(R, 1, NBASES))
    partial, deg = _sc_edge_call(
        table.reshape(R * N, D),
        gidx_p.reshape(E_PAD // CH, CH), tgt_p.reshape(E_PAD // CH, CH))
    return _final_call(partial.reshape(NC, N_PAD, D),
                       deg.reshape(NC, N_PAD, DW),
                       x, self_weight.astype(jnp.float32),
                       bias.astype(jnp.float32).reshape(1, D))
